# baseline (speedup 1.0000x reference)
import functools

import jax
import jax.numpy as jnp
from jax.experimental import pallas as pl
from jax.experimental.pallas import tpu as pltpu

H, S, D = 12, 2048, 64
BLK = 64
NBLK = S // BLK
NSEL = 8
SCALE = 1.0 / (D ** 0.5)
KVX = 72


def _prep_kernel(q_ref, k_ref, v_ref, w_ref, b_ref, sel_ref, kvx_ref, kvt_ref):
    qh = q_ref[0]
    kh = k_ref[0]
    vh = v_ref[0]
    w = w_ref[...]
    b = b_ref[...]

    kproj = jax.lax.dot_general(kh, w, (((1,), (1,)), ((), ())),
                                preferred_element_type=jnp.float32) + b
    kl = jax.nn.softmax(kproj, axis=-1)

    kl_r = kl.reshape(NBLK, BLK, D)
    v_r = vh.reshape(NBLK, BLK, D)
    kv = jax.lax.dot_general(kl_r, v_r, (((1,), (1,)), ((0,), (0,))),
                             preferred_element_type=jnp.float32)
    ksum = jnp.sum(kl_r, axis=1)
    kvx_ref[0, :, 0:D, :] = kv
    kvx_ref[0, :, D:KVX, :] = jnp.broadcast_to(ksum[:, None, :], (NBLK, KVX - D, D))

    kv_tot = jax.lax.dot_general(kl, vh, (((0,), (0,)), ((), ())),
                                 preferred_element_type=jnp.float32)
    ksum_tot = jnp.sum(kl, axis=0)
    kvt_ref[0, 0:D, :] = kv_tot
    kvt_ref[0, D:KVX, :] = jnp.broadcast_to(ksum_tot[None, :], (KVX - D, D))

    qb = jnp.mean(qh.reshape(NBLK, BLK, D), axis=1)
    kb = jnp.mean(kh.reshape(NBLK, BLK, D), axis=1)
    scores = jax.lax.dot_general(qb, kb, (((1,), (1,)), ((), ())),
                                 preferred_element_type=jnp.float32) * SCALE

    colid = jax.lax.broadcasted_iota(jnp.int32, (NBLK, NBLK), 1)
    colid8 = jax.lax.broadcasted_iota(jnp.int32, (NBLK, NSEL), 1)
    cur = scores
    selmat = jnp.zeros((NBLK, NSEL), jnp.int32)
    for t in range(NSEL):
        mx = jnp.max(cur, axis=1, keepdims=True)
        cand = jnp.where(cur == mx, colid, NBLK)
        amin = jnp.min(cand, axis=1, keepdims=True)
        selmat = selmat + amin * (colid8 == t).astype(jnp.int32)
        cur = jnp.where(colid == amin, -1e30, cur)
    sel_ref[0] = selmat


def _attn_kernel(sel_ref, q_ref, k_ref, v_ref, kvx_ref, kvt_ref, w_ref, b_ref,
                 o_ref):
    h = pl.program_id(0)
    qi = pl.program_id(1)
    qb = q_ref[0]
    w = w_ref[...]
    b = b_ref[...]

    qproj = jax.lax.dot_general(qb, w, (((1,), (1,)), ((), ())),
                                preferred_element_type=jnp.float32) + b
    ql = jax.nn.softmax(qproj, axis=-1)

    kvt = kvt_ref[0]
    num = jnp.dot(ql, kvt[0:D, :], preferred_element_type=jnp.float32)
    den = jnp.sum(ql * kvt[D:D + 1, :], axis=1, keepdims=True)

    base = (h * NBLK + qi) * NSEL
    for t in range(NSEL):
        idx = sel_ref[base + t]
        ks = k_ref[0, pl.ds(idx * BLK, BLK), :]
        vs = v_ref[0, pl.ds(idx * BLK, BLK), :]
        s = jax.lax.dot_general(qb, ks, (((1,), (1,)), ((), ())),
                                preferred_element_type=jnp.float32) * SCALE
        p = jnp.exp(s)
        num += jnp.dot(p, vs, preferred_element_type=jnp.float32)
        den += jnp.sum(p, axis=1, keepdims=True)
        kvs = kvx_ref[0, idx]
        num -= jnp.dot(ql, kvs[0:D, :], preferred_element_type=jnp.float32)
        den -= jnp.sum(ql * kvs[D:D + 1, :], axis=1, keepdims=True)

    o_ref[0] = num / (den + 1e-6)


@functools.partial(jax.jit, static_argnames=("interpret",))
def _run(q, k, v, w, b, interpret=False):
    q3 = q[0]
    k3 = k[0]
    v3 = v[0]
    b2 = b.reshape(1, D)

    sel, kvx, kvt = pl.pallas_call(
        _prep_kernel,
        grid=(H,),
        in_specs=[
            pl.BlockSpec((1, S, D), lambda h: (h, 0, 0)),
            pl.BlockSpec((1, S, D), lambda h: (h, 0, 0)),
            pl.BlockSpec((1, S, D), lambda h: (h, 0, 0)),
            pl.BlockSpec((D, D), lambda h: (0, 0)),
            pl.BlockSpec((1, D), lambda h: (0, 0)),
        ],
        out_specs=[
            pl.BlockSpec((1, NBLK, NSEL), lambda h: (h, 0, 0)),
            pl.BlockSpec((1, NBLK, KVX, D), lambda h: (h, 0, 0, 0)),
            pl.BlockSpec((1, KVX, D), lambda h: (h, 0, 0)),
        ],
        out_shape=[
            jax.ShapeDtypeStruct((H, NBLK, NSEL), jnp.int32),
            jax.ShapeDtypeStruct((H, NBLK, KVX, D), jnp.float32),
            jax.ShapeDtypeStruct((H, KVX, D), jnp.float32),
        ],
        interpret=interpret,
    )(q3, k3, v3, w, b2)

    out = pl.pallas_call(
        _attn_kernel,
        grid_spec=pltpu.PrefetchScalarGridSpec(
            num_scalar_prefetch=1,
            grid=(H, NBLK),
            in_specs=[
                pl.BlockSpec((1, BLK, D), lambda h, qi, sel: (h, qi, 0)),
                pl.BlockSpec((1, S, D), lambda h, qi, sel: (h, 0, 0)),
                pl.BlockSpec((1, S, D), lambda h, qi, sel: (h, 0, 0)),
                pl.BlockSpec((1, NBLK, KVX, D), lambda h, qi, sel: (h, 0, 0, 0)),
                pl.BlockSpec((1, KVX, D), lambda h, qi, sel: (h, 0, 0)),
                pl.BlockSpec((D, D), lambda h, qi, sel: (0, 0)),
                pl.BlockSpec((1, D), lambda h, qi, sel: (0, 0)),
            ],
            out_specs=pl.BlockSpec((1, BLK, D), lambda h, qi, sel: (h, qi, 0)),
        ),
        out_shape=jax.ShapeDtypeStruct((H, S, D), jnp.float32),
        interpret=interpret,
    )(sel.reshape(-1), q3, k3, v3, kvx, kvt, w, b2)

    return out.reshape(1, H, S, D)


def kernel(q, k, v, W_l, b_l):
    return _run(q, k, v, W_l, b_l)


# speedup vs baseline: 1.0794x; 1.0794x over previous
import functools

import jax
import jax.numpy as jnp
from jax.experimental import pallas as pl
from jax.experimental.pallas import tpu as pltpu

H, S, D = 12, 2048, 64
BLK = 64
NBLK = S // BLK
NSEL = 8
SCALE = 1.0 / (D ** 0.5)
SG = NSEL * BLK
AW = SG + D
NW = 2 * D


def _prep_kernel(q_ref, k_ref, v_ref, w_ref, b_ref, sel_ref, kvx_ref, kvt_ref,
                 va_ref):
    h = pl.program_id(0)
    qh = q_ref[0]
    kh = k_ref[0]
    vh = v_ref[0]
    w = w_ref[...]
    b = b_ref[...]

    kproj = jax.lax.dot_general(kh, w, (((1,), (1,)), ((), ())),
                                preferred_element_type=jnp.float32) + b
    kl = jax.nn.softmax(kproj, axis=-1)

    @pl.when(h == 0)
    def _init():
        va_ref[:, D:D + 1] = jnp.ones((S, 1), jnp.float32)
        va_ref[:, D + 1:NW] = jnp.zeros((S, NW - D - 1), jnp.float32)

    va_ref[:, 0:D] = vh
    va = va_ref[...]

    kl_r = kl.reshape(NBLK, BLK, D)
    kv = jax.lax.dot_general(kl_r, va.reshape(NBLK, BLK, NW),
                             (((1,), (1,)), ((0,), (0,))),
                             preferred_element_type=jnp.float32)
    kvx_ref[0] = kv
    kvt_ref[0] = jax.lax.dot_general(kl, va, (((0,), (0,)), ((), ())),
                                     preferred_element_type=jnp.float32)

    qb = jnp.mean(qh.reshape(NBLK, BLK, D), axis=1)
    kb = jnp.mean(kh.reshape(NBLK, BLK, D), axis=1)
    scores = jax.lax.dot_general(qb, kb, (((1,), (1,)), ((), ())),
                                 preferred_element_type=jnp.float32) * SCALE

    colid = jax.lax.broadcasted_iota(jnp.int32, (NBLK, NBLK), 1)
    colid8 = jax.lax.broadcasted_iota(jnp.int32, (NBLK, NSEL), 1)
    cur = scores
    selmat = jnp.zeros((NBLK, NSEL), jnp.int32)
    for t in range(NSEL):
        mx = jnp.max(cur, axis=1, keepdims=True)
        cand = jnp.where(cur == mx, colid, NBLK)
        amin = jnp.min(cand, axis=1, keepdims=True)
        selmat = selmat + amin * (colid8 == t).astype(jnp.int32)
        cur = jnp.where(colid == amin, -1e30, cur)
    sel_ref[0] = selmat


def _attn_kernel(sel_ref, q_ref, k_ref, v_ref, kvx_ref, kvt_ref, w_ref, b_ref,
                 o_ref, kg_ref, ag_ref, bg_ref):
    h = pl.program_id(0)
    qi = pl.program_id(1)
    qb = q_ref[0]
    w = w_ref[...]
    b = b_ref[...]

    @pl.when(jnp.logical_and(h == 0, qi == 0))
    def _init():
        bg_ref[0:SG, D:D + 1] = jnp.ones((SG, 1), jnp.float32)

    qproj = jax.lax.dot_general(qb, w, (((1,), (1,)), ((), ())),
                                preferred_element_type=jnp.float32) + b
    ql = jax.nn.softmax(qproj, axis=-1)
    ag_ref[:, SG:AW] = ql

    base = (h * NBLK + qi) * NSEL
    kv_eff = kvt_ref[0]
    for t in range(NSEL):
        idx = sel_ref[base + t]
        kg_ref[pl.ds(t * BLK, BLK), :] = k_ref[0, pl.ds(idx * BLK, BLK), :]
        bg_ref[pl.ds(t * BLK, BLK), 0:D] = v_ref[0, pl.ds(idx * BLK, BLK), :]
        kv_eff = kv_eff - kvx_ref[0, idx]
    bg_ref[SG:AW, :] = kv_eff

    s = jax.lax.dot_general(qb, kg_ref[...], (((1,), (1,)), ((), ())),
                            preferred_element_type=jnp.float32) * SCALE
    ag_ref[:, 0:SG] = jnp.exp(s)

    big = jnp.dot(ag_ref[...], bg_ref[...], preferred_element_type=jnp.float32)
    o_ref[0] = big[:, 0:D] / (big[:, D:D + 1] + 1e-6)


@functools.partial(jax.jit, static_argnames=("interpret",))
def _run(q, k, v, w, b, interpret=False):
    q3 = q[0]
    k3 = k[0]
    v3 = v[0]
    b2 = b.reshape(1, D)

    sel, kvx, kvt = pl.pallas_call(
        _prep_kernel,
        grid=(H,),
        in_specs=[
            pl.BlockSpec((1, S, D), lambda h: (h, 0, 0)),
            pl.BlockSpec((1, S, D), lambda h: (h, 0, 0)),
            pl.BlockSpec((1, S, D), lambda h: (h, 0, 0)),
            pl.BlockSpec((D, D), lambda h: (0, 0)),
            pl.BlockSpec((1, D), lambda h: (0, 0)),
        ],
        out_specs=[
            pl.BlockSpec((1, NBLK, NSEL), lambda h: (h, 0, 0)),
            pl.BlockSpec((1, NBLK, D, NW), lambda h: (h, 0, 0, 0)),
            pl.BlockSpec((1, D, NW), lambda h: (h, 0, 0)),
        ],
        out_shape=[
            jax.ShapeDtypeStruct((H, NBLK, NSEL), jnp.int32),
            jax.ShapeDtypeStruct((H, NBLK, D, NW), jnp.float32),
            jax.ShapeDtypeStruct((H, D, NW), jnp.float32),
        ],
        scratch_shapes=[pltpu.VMEM((S, NW), jnp.float32)],
        interpret=interpret,
    )(q3, k3, v3, w, b2)

    out = pl.pallas_call(
        _attn_kernel,
        grid_spec=pltpu.PrefetchScalarGridSpec(
            num_scalar_prefetch=1,
            grid=(H, NBLK),
            in_specs=[
                pl.BlockSpec((1, BLK, D), lambda h, qi, sel: (h, qi, 0)),
                pl.BlockSpec((1, S, D), lambda h, qi, sel: (h, 0, 0)),
                pl.BlockSpec((1, S, D), lambda h, qi, sel: (h, 0, 0)),
                pl.BlockSpec((1, NBLK, D, NW), lambda h, qi, sel: (h, 0, 0, 0)),
                pl.BlockSpec((1, D, NW), lambda h, qi, sel: (h, 0, 0)),
                pl.BlockSpec((D, D), lambda h, qi, sel: (0, 0)),
                pl.BlockSpec((1, D), lambda h, qi, sel: (0, 0)),
            ],
            out_specs=pl.BlockSpec((1, BLK, D), lambda h, qi, sel: (h, qi, 0)),
            scratch_shapes=[
                pltpu.VMEM((SG, D), jnp.float32),
                pltpu.VMEM((BLK, AW), jnp.float32),
                pltpu.VMEM((AW, NW), jnp.float32),
            ],
        ),
        out_shape=jax.ShapeDtypeStruct((H, S, D), jnp.float32),
        interpret=interpret,
    )(sel.reshape(-1), q3, k3, v3, kvx, kvt, w, b2)

    return out.reshape(1, H, S, D)


def kernel(q, k, v, W_l, b_l):
    return _run(q, k, v, W_l, b_l)


# speedup vs baseline: 1.7394x; 1.6115x over previous
import functools

import jax
import jax.numpy as jnp
from jax.experimental import pallas as pl
from jax.experimental.pallas import tpu as pltpu

H, S, D = 12, 2048, 64
BLK = 64
NBLK = S // BLK
NSEL = 8
SCALE = 1.0 / (D ** 0.5)
SG = NSEL * BLK
NW = 2 * D
QB = 4
QG = NBLK // QB


def _prep_kernel(q_ref, k_ref, v_ref, w_ref, b_ref, sel_ref, kvx_ref, kvt_ref,
                 va_ref):
    h = pl.program_id(0)
    qh = q_ref[0]
    kh = k_ref[0]
    vh = v_ref[0]
    w = w_ref[...]
    b = b_ref[...]

    kproj = jax.lax.dot_general(kh, w, (((1,), (1,)), ((), ())),
                                preferred_element_type=jnp.float32) + b
    kl = jax.nn.softmax(kproj, axis=-1)

    @pl.when(h == 0)
    def _init():
        va_ref[:, D:D + 1] = jnp.ones((S, 1), jnp.float32)
        va_ref[:, D + 1:NW] = jnp.zeros((S, NW - D - 1), jnp.float32)

    va_ref[:, 0:D] = vh
    va = va_ref[...]

    kl_r = kl.reshape(NBLK, BLK, D)
    kv = jax.lax.dot_general(kl_r, va.reshape(NBLK, BLK, NW),
                             (((1,), (1,)), ((0,), (0,))),
                             preferred_element_type=jnp.float32)
    kvx_ref[0] = kv
    kvt_ref[0] = jax.lax.dot_general(kl, va, (((0,), (0,)), ((), ())),
                                     preferred_element_type=jnp.float32)

    qb = jnp.mean(qh.reshape(NBLK, BLK, D), axis=1)
    kb = jnp.mean(kh.reshape(NBLK, BLK, D), axis=1)
    scores = jax.lax.dot_general(qb, kb, (((1,), (1,)), ((), ())),
                                 preferred_element_type=jnp.float32) * SCALE

    colid = jax.lax.broadcasted_iota(jnp.int32, (NBLK, NBLK), 1)
    colid8 = jax.lax.broadcasted_iota(jnp.int32, (NBLK, NSEL), 1)
    cur = scores
    selmat = jnp.zeros((NBLK, NSEL), jnp.int32)
    for t in range(NSEL):
        mx = jnp.max(cur, axis=1, keepdims=True)
        cand = jnp.where(cur == mx, colid, NBLK)
        amin = jnp.min(cand, axis=1, keepdims=True)
        selmat = selmat + amin * (colid8 == t).astype(jnp.int32)
        cur = jnp.where(colid == amin, -1e30, cur)
    sel_ref[0] = selmat


def _attn_kernel(sel_ref, q_ref, k_ref, v_ref, kvx_ref, kvt_ref, w_ref, b_ref,
                 o_ref, kg_ref, vg_ref):
    h = pl.program_id(0)
    qi = pl.program_id(1)
    w = w_ref[...]
    b = b_ref[...]

    @pl.when(jnp.logical_and(h == 0, qi == 0))
    def _init():
        for j in range(QB):
            vg_ref[j, :, D:D + 1] = jnp.ones((SG, 1), jnp.float32)

    kvt = kvt_ref[0]
    for j in range(QB):
        base = (h * NBLK + qi * QB + j) * NSEL
        kv_eff = kvt
        for t in range(NSEL):
            idx = sel_ref[base + t]
            kg_ref[j, pl.ds(t * BLK, BLK), :] = k_ref[0, pl.ds(idx * BLK, BLK), :]
            vg_ref[j, pl.ds(t * BLK, BLK), 0:D] = v_ref[0, pl.ds(idx * BLK, BLK), :]
            kv_eff = kv_eff - kvx_ref[0, idx]

        qb = q_ref[0, pl.ds(j * BLK, BLK), :]
        qproj = jax.lax.dot_general(qb, w, (((1,), (1,)), ((), ())),
                                    preferred_element_type=jnp.float32) + b
        ql = jax.nn.softmax(qproj, axis=-1)

        s = jax.lax.dot_general(qb, kg_ref[j], (((1,), (1,)), ((), ())),
                                preferred_element_type=jnp.float32) * SCALE
        p = jnp.exp(s)
        big = jnp.dot(p, vg_ref[j], preferred_element_type=jnp.float32)
        big = big + jnp.dot(ql, kv_eff, preferred_element_type=jnp.float32)
        o_ref[0, pl.ds(j * BLK, BLK), :] = (
            big[:, 0:D] / (big[:, D:D + 1] + 1e-6))


@functools.partial(jax.jit, static_argnames=("interpret",))
def _run(q, k, v, w, b, interpret=False):
    q3 = q[0]
    k3 = k[0]
    v3 = v[0]
    b2 = b.reshape(1, D)

    sel, kvx, kvt = pl.pallas_call(
        _prep_kernel,
        grid=(H,),
        in_specs=[
            pl.BlockSpec((1, S, D), lambda h: (h, 0, 0)),
            pl.BlockSpec((1, S, D), lambda h: (h, 0, 0)),
            pl.BlockSpec((1, S, D), lambda h: (h, 0, 0)),
            pl.BlockSpec((D, D), lambda h: (0, 0)),
            pl.BlockSpec((1, D), lambda h: (0, 0)),
        ],
        out_specs=[
            pl.BlockSpec((1, NBLK, NSEL), lambda h: (h, 0, 0)),
            pl.BlockSpec((1, NBLK, D, NW), lambda h: (h, 0, 0, 0)),
            pl.BlockSpec((1, D, NW), lambda h: (h, 0, 0)),
        ],
        out_shape=[
            jax.ShapeDtypeStruct((H, NBLK, NSEL), jnp.int32),
            jax.ShapeDtypeStruct((H, NBLK, D, NW), jnp.float32),
            jax.ShapeDtypeStruct((H, D, NW), jnp.float32),
        ],
        scratch_shapes=[pltpu.VMEM((S, NW), jnp.float32)],
        interpret=interpret,
    )(q3, k3, v3, w, b2)

    out = pl.pallas_call(
        _attn_kernel,
        grid_spec=pltpu.PrefetchScalarGridSpec(
            num_scalar_prefetch=1,
            grid=(H, QG),
            in_specs=[
                pl.BlockSpec((1, QB * BLK, D), lambda h, qi, sel: (h, qi, 0)),
                pl.BlockSpec((1, S, D), lambda h, qi, sel: (h, 0, 0)),
                pl.BlockSpec((1, S, D), lambda h, qi, sel: (h, 0, 0)),
                pl.BlockSpec((1, NBLK, D, NW), lambda h, qi, sel: (h, 0, 0, 0)),
                pl.BlockSpec((1, D, NW), lambda h, qi, sel: (h, 0, 0)),
                pl.BlockSpec((D, D), lambda h, qi, sel: (0, 0)),
                pl.BlockSpec((1, D), lambda h, qi, sel: (0, 0)),
            ],
            out_specs=pl.BlockSpec((1, QB * BLK, D), lambda h, qi, sel: (h, qi, 0)),
            scratch_shapes=[
                pltpu.VMEM((QB, SG, D), jnp.float32),
                pltpu.VMEM((QB, SG, NW), jnp.float32),
            ],
        ),
        out_shape=jax.ShapeDtypeStruct((H, S, D), jnp.float32),
        interpret=interpret,
    )(sel.reshape(-1), q3, k3, v3, kvx, kvt, w, b2)

    return out.reshape(1, H, S, D)


def kernel(q, k, v, W_l, b_l):
    return _run(q, k, v, W_l, b_l)


# speedup vs baseline: 1.8356x; 1.0553x over previous
import functools

import jax
import jax.numpy as jnp
from jax.experimental import pallas as pl
from jax.experimental.pallas import tpu as pltpu

H, S, D = 12, 2048, 64
BLK = 64
NBLK = S // BLK
NSEL = 8
SCALE = 1.0 / (D ** 0.5)
SG = NSEL * BLK
NW = 2 * D
QB = 4
QG = NBLK // QB


def _prep_kernel(q_ref, k_ref, v_ref, w_ref, b_ref, sel_ref, kvx_ref, kvt_ref,
                 va_ref):
    h = pl.program_id(0)
    qh = q_ref[0, 0]
    kh = k_ref[0, 0]
    vh = v_ref[0, 0]
    w = w_ref[...]
    b = b_ref[...]

    kproj = jax.lax.dot_general(kh, w, (((1,), (1,)), ((), ())),
                                preferred_element_type=jnp.float32) + b
    kl = jax.nn.softmax(kproj, axis=-1)

    @pl.when(h == 0)
    def _init():
        va_ref[:, D:D + 1] = jnp.ones((S, 1), jnp.float32)
        va_ref[:, D + 1:NW] = jnp.zeros((S, NW - D - 1), jnp.float32)

    va_ref[:, 0:D] = vh
    va = va_ref[...]

    kl_r = kl.reshape(NBLK, BLK, D)
    kv = jax.lax.dot_general(kl_r, va.reshape(NBLK, BLK, NW),
                             (((1,), (1,)), ((0,), (0,))),
                             preferred_element_type=jnp.float32)
    kvx_ref[0] = kv
    kvt_ref[0] = jax.lax.dot_general(kl, va, (((0,), (0,)), ((), ())),
                                     preferred_element_type=jnp.float32)

    qb = jnp.mean(qh.reshape(NBLK, BLK, D), axis=1)
    kb = jnp.mean(kh.reshape(NBLK, BLK, D), axis=1)
    scores = jax.lax.dot_general(qb, kb, (((1,), (1,)), ((), ())),
                                 preferred_element_type=jnp.float32) * SCALE

    colid = jax.lax.broadcasted_iota(jnp.int32, (NBLK, NBLK), 1)
    colid8 = jax.lax.broadcasted_iota(jnp.int32, (NBLK, NSEL), 1)
    cur = scores
    selmat = jnp.zeros((NBLK, NSEL), jnp.int32)
    for t in range(NSEL):
        mx = jnp.max(cur, axis=1, keepdims=True)
        cand = jnp.where(cur == mx, colid, NBLK)
        amin = jnp.min(cand, axis=1, keepdims=True)
        selmat = selmat + amin * (colid8 == t).astype(jnp.int32)
        cur = jnp.where(colid == amin, -1e30, cur)
    sel_ref[0] = selmat


def _attn_kernel(sel_ref, q_ref, k_ref, v_ref, kvx_ref, kvt_ref, w_ref, b_ref,
                 o_ref, kg_ref, vg_ref):
    h = pl.program_id(0)
    qi = pl.program_id(1)
    w = w_ref[...]
    b = b_ref[...]

    @pl.when(jnp.logical_and(h == 0, qi == 0))
    def _init():
        for j in range(QB):
            vg_ref[j, :, D:D + 1] = jnp.ones((SG, 1), jnp.float32)

    kvt = kvt_ref[0]
    for j in range(QB):
        base = (h * NBLK + qi * QB + j) * NSEL
        kv_eff = kvt
        for t in range(NSEL):
            idx = sel_ref[base + t]
            kg_ref[j, pl.ds(t * BLK, BLK), :] = k_ref[0, 0, pl.ds(idx * BLK, BLK), :]
            vg_ref[j, pl.ds(t * BLK, BLK), 0:D] = v_ref[0, 0, pl.ds(idx * BLK, BLK), :]
            kv_eff = kv_eff - kvx_ref[0, idx]

        qb = q_ref[0, 0, pl.ds(j * BLK, BLK), :]
        qproj = jax.lax.dot_general(qb, w, (((1,), (1,)), ((), ())),
                                    preferred_element_type=jnp.float32) + b
        ql = jax.nn.softmax(qproj, axis=-1)

        s = jax.lax.dot_general(qb, kg_ref[j], (((1,), (1,)), ((), ())),
                                preferred_element_type=jnp.float32) * SCALE
        p = jnp.exp(s)
        big = jnp.dot(p, vg_ref[j], preferred_element_type=jnp.float32)
        big = big + jnp.dot(ql, kv_eff, preferred_element_type=jnp.float32)
        o_ref[0, 0, pl.ds(j * BLK, BLK), :] = (
            big[:, 0:D] / (big[:, D:D + 1] + 1e-6))


@functools.partial(jax.jit, static_argnames=("interpret",))
def _run(q, k, v, w, b, interpret=False):
    b2 = b.reshape(1, D)

    sel, kvx, kvt = pl.pallas_call(
        _prep_kernel,
        grid=(H,),
        in_specs=[
            pl.BlockSpec((1, 1, S, D), lambda h: (0, h, 0, 0)),
            pl.BlockSpec((1, 1, S, D), lambda h: (0, h, 0, 0)),
            pl.BlockSpec((1, 1, S, D), lambda h: (0, h, 0, 0)),
            pl.BlockSpec((D, D), lambda h: (0, 0)),
            pl.BlockSpec((1, D), lambda h: (0, 0)),
        ],
        out_specs=[
            pl.BlockSpec((1, NBLK, NSEL), lambda h: (h, 0, 0)),
            pl.BlockSpec((1, NBLK, D, NW), lambda h: (h, 0, 0, 0)),
            pl.BlockSpec((1, D, NW), lambda h: (h, 0, 0)),
        ],
        out_shape=[
            jax.ShapeDtypeStruct((H, NBLK, NSEL), jnp.int32),
            jax.ShapeDtypeStruct((H, NBLK, D, NW), jnp.float32),
            jax.ShapeDtypeStruct((H, D, NW), jnp.float32),
        ],
        scratch_shapes=[pltpu.VMEM((S, NW), jnp.float32)],
        interpret=interpret,
    )(q, k, v, w, b2)

    out = pl.pallas_call(
        _attn_kernel,
        grid_spec=pltpu.PrefetchScalarGridSpec(
            num_scalar_prefetch=1,
            grid=(H, QG),
            in_specs=[
                pl.BlockSpec((1, 1, QB * BLK, D), lambda h, qi, sel: (0, h, qi, 0)),
                pl.BlockSpec((1, 1, S, D), lambda h, qi, sel: (0, h, 0, 0)),
                pl.BlockSpec((1, 1, S, D), lambda h, qi, sel: (0, h, 0, 0)),
                pl.BlockSpec((1, NBLK, D, NW), lambda h, qi, sel: (h, 0, 0, 0)),
                pl.BlockSpec((1, D, NW), lambda h, qi, sel: (h, 0, 0)),
                pl.BlockSpec((D, D), lambda h, qi, sel: (0, 0)),
                pl.BlockSpec((1, D), lambda h, qi, sel: (0, 0)),
            ],
            out_specs=pl.BlockSpec((1, 1, QB * BLK, D),
                                   lambda h, qi, sel: (0, h, qi, 0)),
            scratch_shapes=[
                pltpu.VMEM((QB, SG, D), jnp.float32),
                pltpu.VMEM((QB, SG, NW), jnp.float32),
            ],
        ),
        out_shape=jax.ShapeDtypeStruct((1, H, S, D), jnp.float32),
        interpret=interpret,
    )(sel.reshape(-1), q, k, v, kvx, kvt, w, b2)

    return out


def kernel(q, k, v, W_l, b_l):
    return _run(q, k, v, W_l, b_l)


# speedup vs baseline: 1.9984x; 1.0887x over previous
import functools

import jax
import jax.numpy as jnp
from jax.experimental import pallas as pl
from jax.experimental.pallas import tpu as pltpu

H, S, D = 12, 2048, 64
BLK = 64
NBLK = S // BLK
NSEL = 8
SCALE = 1.0 / (D ** 0.5)
SG = NSEL * BLK
NW = 2 * D
QB = 8
QG = NBLK // QB


def _prep_kernel(q_ref, k_ref, v_ref, w_ref, b_ref, sel_ref, kvx_ref, kvt_ref,
                 va_ref):
    h = pl.program_id(0)
    qh = q_ref[0, 0]
    kh = k_ref[0, 0]
    vh = v_ref[0, 0]
    w = w_ref[...]
    b = b_ref[...]

    kproj = jax.lax.dot_general(kh.astype(jnp.bfloat16), w.astype(jnp.bfloat16),
                                (((1,), (1,)), ((), ())),
                                preferred_element_type=jnp.float32) + b
    kl = jax.nn.softmax(kproj, axis=-1)

    @pl.when(h == 0)
    def _init():
        va_ref[:, D:D + 1] = jnp.ones((S, 1), jnp.float32)
        va_ref[:, D + 1:NW] = jnp.zeros((S, NW - D - 1), jnp.float32)

    va_ref[:, 0:D] = vh
    va = va_ref[...]

    kl16 = kl.astype(jnp.bfloat16)
    va16 = va.astype(jnp.bfloat16)
    kl_r = kl16.reshape(NBLK, BLK, D)
    kv = jax.lax.dot_general(kl_r, va16.reshape(NBLK, BLK, NW),
                             (((1,), (1,)), ((0,), (0,))),
                             preferred_element_type=jnp.float32)
    kvx_ref[0] = kv
    kvt_ref[0] = jax.lax.dot_general(kl16, va16, (((0,), (0,)), ((), ())),
                                     preferred_element_type=jnp.float32)

    qb = jnp.mean(qh.reshape(NBLK, BLK, D), axis=1)
    kb = jnp.mean(kh.reshape(NBLK, BLK, D), axis=1)
    scores = jax.lax.dot_general(qb, kb, (((1,), (1,)), ((), ())),
                                 preferred_element_type=jnp.float32) * SCALE

    colid = jax.lax.broadcasted_iota(jnp.int32, (NBLK, NBLK), 1)
    colid8 = jax.lax.broadcasted_iota(jnp.int32, (NBLK, NSEL), 1)
    cur = scores
    selmat = jnp.zeros((NBLK, NSEL), jnp.int32)
    for t in range(NSEL):
        mx = jnp.max(cur, axis=1, keepdims=True)
        cand = jnp.where(cur == mx, colid, NBLK)
        amin = jnp.min(cand, axis=1, keepdims=True)
        selmat = selmat + amin * (colid8 == t).astype(jnp.int32)
        cur = jnp.where(colid == amin, -1e30, cur)
    sel_ref[0] = selmat


def _attn_kernel(sel_ref, q_ref, k_ref, v_ref, kvx_ref, kvt_ref, w_ref, b_ref,
                 o_ref, kg_ref, vg_ref):
    h = pl.program_id(0)
    qi = pl.program_id(1)
    w = w_ref[...]
    b = b_ref[...]

    @pl.when(jnp.logical_and(h == 0, qi == 0))
    def _init():
        for j in range(QB):
            vg_ref[j, :, D:D + 1] = jnp.ones((SG, 1), jnp.bfloat16)

    kvt = kvt_ref[0]
    for j in range(QB):
        base = (h * NBLK + qi * QB + j) * NSEL
        kv_eff = kvt
        for t in range(NSEL):
            idx = sel_ref[base + t]
            kg_ref[j, pl.ds(t * BLK, BLK), :] = k_ref[0, 0, pl.ds(idx * BLK, BLK), :].astype(jnp.bfloat16)
            vg_ref[j, pl.ds(t * BLK, BLK), 0:D] = v_ref[0, 0, pl.ds(idx * BLK, BLK), :].astype(jnp.bfloat16)
            kv_eff = kv_eff - kvx_ref[0, idx]

        qb = q_ref[0, 0, pl.ds(j * BLK, BLK), :]
        qproj = jax.lax.dot_general(qb.astype(jnp.bfloat16),
                                    w.astype(jnp.bfloat16),
                                    (((1,), (1,)), ((), ())),
                                    preferred_element_type=jnp.float32) + b
        ql = jax.nn.softmax(qproj, axis=-1)

        s = jax.lax.dot_general(qb.astype(jnp.bfloat16), kg_ref[j],
                                (((1,), (1,)), ((), ())),
                                preferred_element_type=jnp.float32) * SCALE
        p = jnp.exp(s)
        big = jnp.dot(p.astype(jnp.bfloat16), vg_ref[j],
                      preferred_element_type=jnp.float32)
        big = big + jnp.dot(ql.astype(jnp.bfloat16), kv_eff.astype(jnp.bfloat16),
                            preferred_element_type=jnp.float32)
        o_ref[0, 0, pl.ds(j * BLK, BLK), :] = (
            big[:, 0:D] / (big[:, D:D + 1] + 1e-6))


@functools.partial(jax.jit, static_argnames=("interpret",))
def _run(q, k, v, w, b, interpret=False):
    b2 = b.reshape(1, D)

    sel, kvx, kvt = pl.pallas_call(
        _prep_kernel,
        grid=(H,),
        in_specs=[
            pl.BlockSpec((1, 1, S, D), lambda h: (0, h, 0, 0)),
            pl.BlockSpec((1, 1, S, D), lambda h: (0, h, 0, 0)),
            pl.BlockSpec((1, 1, S, D), lambda h: (0, h, 0, 0)),
            pl.BlockSpec((D, D), lambda h: (0, 0)),
            pl.BlockSpec((1, D), lambda h: (0, 0)),
        ],
        out_specs=[
            pl.BlockSpec((1, NBLK, NSEL), lambda h: (h, 0, 0)),
            pl.BlockSpec((1, NBLK, D, NW), lambda h: (h, 0, 0, 0)),
            pl.BlockSpec((1, D, NW), lambda h: (h, 0, 0)),
        ],
        out_shape=[
            jax.ShapeDtypeStruct((H, NBLK, NSEL), jnp.int32),
            jax.ShapeDtypeStruct((H, NBLK, D, NW), jnp.float32),
            jax.ShapeDtypeStruct((H, D, NW), jnp.float32),
        ],
        scratch_shapes=[pltpu.VMEM((S, NW), jnp.float32)],
        interpret=interpret,
    )(q, k, v, w, b2)

    out = pl.pallas_call(
        _attn_kernel,
        grid_spec=pltpu.PrefetchScalarGridSpec(
            num_scalar_prefetch=1,
            grid=(H, QG),
            in_specs=[
                pl.BlockSpec((1, 1, QB * BLK, D), lambda h, qi, sel: (0, h, qi, 0)),
                pl.BlockSpec((1, 1, S, D), lambda h, qi, sel: (0, h, 0, 0)),
                pl.BlockSpec((1, 1, S, D), lambda h, qi, sel: (0, h, 0, 0)),
                pl.BlockSpec((1, NBLK, D, NW), lambda h, qi, sel: (h, 0, 0, 0)),
                pl.BlockSpec((1, D, NW), lambda h, qi, sel: (h, 0, 0)),
                pl.BlockSpec((D, D), lambda h, qi, sel: (0, 0)),
                pl.BlockSpec((1, D), lambda h, qi, sel: (0, 0)),
            ],
            out_specs=pl.BlockSpec((1, 1, QB * BLK, D),
                                   lambda h, qi, sel: (0, h, qi, 0)),
            scratch_shapes=[
                pltpu.VMEM((QB, SG, D), jnp.bfloat16),
                pltpu.VMEM((QB, SG, NW), jnp.bfloat16),
            ],
        ),
        out_shape=jax.ShapeDtypeStruct((1, H, S, D), jnp.float32),
        interpret=interpret,
    )(sel.reshape(-1), q, k, v, kvx, kvt, w, b2)

    return out


def kernel(q, k, v, W_l, b_l):
    return _run(q, k, v, W_l, b_l)


# speedup vs baseline: 2.3435x; 1.1726x over previous
import functools

import jax
import jax.numpy as jnp
from jax.experimental import pallas as pl
from jax.experimental.pallas import tpu as pltpu

H, S, D = 12, 2048, 64
BLK = 64
NBLK = S // BLK
NSEL = 8
SCALE = 1.0 / (D ** 0.5)
SG = NSEL * BLK
NW = 2 * D
QB = 8
QG = NBLK // QB


def _prep_kernel(qt_ref, kt_ref, vt_ref, w_ref, bt_ref,
                 sel_ref, k16_ref, va16_ref, kvx_ref, kvt_ref, bo_ref):
    h = pl.program_id(0)
    qt = qt_ref[0, 0]
    kt = kt_ref[0, 0]
    vt = vt_ref[0, 0]
    w = w_ref[...]
    b = bt_ref[...]

    k16_ref[0] = jnp.transpose(kt.astype(jnp.bfloat16))
    va16_ref[0, :, 0:D] = jnp.transpose(vt.astype(jnp.bfloat16))
    va16_ref[0, :, D:D + 1] = jnp.ones((S, 1), jnp.bfloat16)
    va16_ref[0, :, D + 1:NW] = jnp.zeros((S, NW - D - 1), jnp.bfloat16)

    kproj_t = jax.lax.dot_general(w.astype(jnp.bfloat16),
                                  kt.astype(jnp.bfloat16),
                                  (((1,), (0,)), ((), ())),
                                  preferred_element_type=jnp.float32)
    kproj_t = kproj_t + b
    mx = jnp.max(kproj_t, axis=0, keepdims=True)
    ex = jnp.exp(kproj_t - mx)
    klt = (ex / jnp.sum(ex, axis=0, keepdims=True)).astype(jnp.bfloat16)

    va16 = va16_ref[0]
    for n in range(NBLK):
        kvx_ref[0, n] = jax.lax.dot_general(
            klt[:, n * BLK:(n + 1) * BLK], va16[n * BLK:(n + 1) * BLK, :],
            (((1,), (0,)), ((), ())),
            preferred_element_type=jnp.float32).astype(jnp.bfloat16)
    kvt_ref[0] = jax.lax.dot_general(klt, va16, (((1,), (0,)), ((), ())),
                                     preferred_element_type=jnp.float32)

    @pl.when(h == 0)
    def _init():
        r = jax.lax.broadcasted_iota(jnp.int32, (S, NBLK), 0)
        c = jax.lax.broadcasted_iota(jnp.int32, (S, NBLK), 1)
        bo_ref[...] = jnp.where(r // BLK == c, 1.0 / BLK, 0.0)

    bo = bo_ref[...]
    qbt = jax.lax.dot_general(qt, bo, (((1,), (0,)), ((), ())),
                              preferred_element_type=jnp.float32)
    kbt = jax.lax.dot_general(kt, bo, (((1,), (0,)), ((), ())),
                              preferred_element_type=jnp.float32)
    scores = jax.lax.dot_general(qbt, kbt, (((0,), (0,)), ((), ())),
                                 preferred_element_type=jnp.float32) * SCALE

    colid = jax.lax.broadcasted_iota(jnp.int32, (NBLK, NBLK), 1)
    colid8 = jax.lax.broadcasted_iota(jnp.int32, (NBLK, NSEL), 1)
    cur = scores
    selmat = jnp.zeros((NBLK, NSEL), jnp.int32)
    for t in range(NSEL):
        mx2 = jnp.max(cur, axis=1, keepdims=True)
        cand = jnp.where(cur == mx2, colid, NBLK)
        amin = jnp.min(cand, axis=1, keepdims=True)
        selmat = selmat + amin * (colid8 == t).astype(jnp.int32)
        cur = jnp.where(colid == amin, -1e30, cur)
    sel_ref[0] = selmat


def _attn_kernel(sel_ref, qt_ref, k16_ref, va16_ref, kvx_ref, kvt_ref,
                 w_ref, b_ref, o_ref, kg_ref, vg_ref):
    h = pl.program_id(0)
    qi = pl.program_id(1)
    w16 = w_ref[...].astype(jnp.bfloat16)
    b = b_ref[...]

    kvt = kvt_ref[0]
    for j in range(QB):
        base = (h * NBLK + qi * QB + j) * NSEL
        kv_eff = kvt
        for t in range(NSEL):
            idx = sel_ref[base + t]
            kg_ref[j, pl.ds(t * BLK, BLK), :] = k16_ref[0, pl.ds(idx * BLK, BLK), :]
            vg_ref[j, pl.ds(t * BLK, BLK), :] = va16_ref[0, pl.ds(idx * BLK, BLK), :]
            kv_eff = kv_eff - kvx_ref[0, idx].astype(jnp.float32)

        qtj = qt_ref[0, 0, :, pl.ds(j * BLK, BLK)].astype(jnp.bfloat16)
        qproj = jax.lax.dot_general(qtj, w16, (((0,), (1,)), ((), ())),
                                    preferred_element_type=jnp.float32) + b
        ql = jax.nn.softmax(qproj, axis=-1)

        s = jax.lax.dot_general(qtj, kg_ref[j], (((0,), (1,)), ((), ())),
                                preferred_element_type=jnp.float32) * SCALE
        p = jnp.exp(s)
        big = jnp.dot(p.astype(jnp.bfloat16), vg_ref[j],
                      preferred_element_type=jnp.float32)
        big = big + jnp.dot(ql.astype(jnp.bfloat16), kv_eff.astype(jnp.bfloat16),
                            preferred_element_type=jnp.float32)
        o_ref[0, 0, pl.ds(j * BLK, BLK), :] = (
            big[:, 0:D] / (big[:, D:D + 1] + 1e-6))


@functools.partial(jax.jit, static_argnames=("interpret",))
def _run(q, k, v, w, b, interpret=False):
    b2 = b.reshape(1, D)
    bt = b.reshape(D, 1)
    qt = jnp.transpose(q, (0, 1, 3, 2))
    kt = jnp.transpose(k, (0, 1, 3, 2))
    vt = jnp.transpose(v, (0, 1, 3, 2))

    sel, k16, va16, kvx, kvt = pl.pallas_call(
        _prep_kernel,
        grid=(H,),
        in_specs=[
            pl.BlockSpec((1, 1, D, S), lambda h: (0, h, 0, 0)),
            pl.BlockSpec((1, 1, D, S), lambda h: (0, h, 0, 0)),
            pl.BlockSpec((1, 1, D, S), lambda h: (0, h, 0, 0)),
            pl.BlockSpec((D, D), lambda h: (0, 0)),
            pl.BlockSpec((D, 1), lambda h: (0, 0)),
        ],
        out_specs=[
            pl.BlockSpec((1, NBLK, NSEL), lambda h: (h, 0, 0)),
            pl.BlockSpec((1, S, D), lambda h: (h, 0, 0)),
            pl.BlockSpec((1, S, NW), lambda h: (h, 0, 0)),
            pl.BlockSpec((1, NBLK, D, NW), lambda h: (h, 0, 0, 0)),
            pl.BlockSpec((1, D, NW), lambda h: (h, 0, 0)),
        ],
        out_shape=[
            jax.ShapeDtypeStruct((H, NBLK, NSEL), jnp.int32),
            jax.ShapeDtypeStruct((H, S, D), jnp.bfloat16),
            jax.ShapeDtypeStruct((H, S, NW), jnp.bfloat16),
            jax.ShapeDtypeStruct((H, NBLK, D, NW), jnp.bfloat16),
            jax.ShapeDtypeStruct((H, D, NW), jnp.float32),
        ],
        scratch_shapes=[pltpu.VMEM((S, NBLK), jnp.float32)],
        interpret=interpret,
    )(qt, kt, vt, w, bt)

    out = pl.pallas_call(
        _attn_kernel,
        grid_spec=pltpu.PrefetchScalarGridSpec(
            num_scalar_prefetch=1,
            grid=(H, QG),
            in_specs=[
                pl.BlockSpec((1, 1, D, QB * BLK), lambda h, qi, sel: (0, h, 0, qi)),
                pl.BlockSpec((1, S, D), lambda h, qi, sel: (h, 0, 0)),
                pl.BlockSpec((1, S, NW), lambda h, qi, sel: (h, 0, 0)),
                pl.BlockSpec((1, NBLK, D, NW), lambda h, qi, sel: (h, 0, 0, 0)),
                pl.BlockSpec((1, D, NW), lambda h, qi, sel: (h, 0, 0)),
                pl.BlockSpec((D, D), lambda h, qi, sel: (0, 0)),
                pl.BlockSpec((1, D), lambda h, qi, sel: (0, 0)),
            ],
            out_specs=pl.BlockSpec((1, 1, QB * BLK, D),
                                   lambda h, qi, sel: (0, h, qi, 0)),
            scratch_shapes=[
                pltpu.VMEM((QB, SG, D), jnp.bfloat16),
                pltpu.VMEM((QB, SG, NW), jnp.bfloat16),
            ],
        ),
        out_shape=jax.ShapeDtypeStruct((1, H, S, D), jnp.float32),
        interpret=interpret,
    )(sel.reshape(-1), qt, k16, va16, kvx, kvt, w, b2)

    return out


def kernel(q, k, v, W_l, b_l):
    return _run(q, k, v, W_l, b_l)
